# baseline (speedup 1.0000x reference)
import functools

import jax
import jax.numpy as jnp
from jax import lax
from jax.experimental import pallas as pl
from jax.experimental.pallas import tpu as pltpu
from jax.experimental.pallas import tpu_sc as plsc

EMBED_DIM = 300
NUM_CORES = 2
NUM_SUBCORES = 16
NUM_WORKERS = NUM_CORES * NUM_SUBCORES
CHUNK = 128

_mesh = plsc.VectorSubcoreMesh(core_axis_name="c", subcore_axis_name="s")


def _make_gather(n_chunks: int, b_per_w: int, total_rows: int):
    @functools.partial(
        pl.kernel,
        mesh=_mesh,
        out_type=jax.ShapeDtypeStruct((total_rows, EMBED_DIM), jnp.float32),
        scratch_types=[
            pltpu.VMEM((n_chunks, CHUNK), jnp.int32),
            pltpu.VMEM((CHUNK, EMBED_DIM), jnp.float32),
            pltpu.SemaphoreType.DMA,
        ],
        compiler_params=pltpu.CompilerParams(use_tc_tiling_on_sc=False),
    )
    def gather(idx_hbm, table_hbm, out_hbm, idx_v, rows_v, sem):
        wid = lax.axis_index("s") * NUM_CORES + lax.axis_index("c")
        pltpu.sync_copy(idx_hbm.at[wid], idx_v)
        base = wid * b_per_w

        def chunk_body(c, carry):
            pltpu.async_copy(table_hbm.at[idx_v.at[c]], rows_v, sem).wait()
            pltpu.sync_copy(rows_v, out_hbm.at[pl.ds(base + c * CHUNK, CHUNK)])
            return carry

        lax.fori_loop(0, n_chunks, chunk_body, 0)

    return gather


def kernel(indices, table):
    batch, seq = indices.shape
    total = batch * seq
    b_per_w = total // NUM_WORKERS
    n_chunks = b_per_w // CHUNK
    idx = indices.reshape(-1).astype(jnp.int32).reshape(
        NUM_WORKERS, n_chunks, CHUNK)
    out = _make_gather(n_chunks, b_per_w, total)(idx, table)
    return out.reshape(batch, seq, EMBED_DIM)


# speedup vs baseline: 1.0434x; 1.0434x over previous
import functools

import jax
import jax.numpy as jnp
from jax import lax
from jax.experimental import pallas as pl
from jax.experimental.pallas import tpu as pltpu
from jax.experimental.pallas import tpu_sc as plsc

EMBED_DIM = 300
PAIR_DIM = 2 * EMBED_DIM
NUM_CORES = 2
NUM_SUBCORES = 16
NUM_WORKERS = NUM_CORES * NUM_SUBCORES
CHUNK = 128
LANES = 16

_mesh = plsc.VectorSubcoreMesh(core_axis_name="c", subcore_axis_name="s")


def _make_gather(n_chunks: int, b_per_w: int, total_rows: int):
    @functools.partial(
        pl.kernel,
        mesh=_mesh,
        out_type=jax.ShapeDtypeStruct((total_rows * EMBED_DIM,), jnp.float32),
        scratch_types=[
            pltpu.VMEM((b_per_w,), jnp.int32),
            pltpu.VMEM((b_per_w,), jnp.int32),
            pltpu.VMEM((CHUNK, PAIR_DIM), jnp.float32),
            pltpu.VMEM((CHUNK * EMBED_DIM,), jnp.float32),
            pltpu.SemaphoreType.DMA,
        ],
        compiler_params=pltpu.CompilerParams(
            use_tc_tiling_on_sc=False, needs_layout_passes=False),
    )
    def gather(pidx_hbm, soff_hbm, tpair_hbm, out_hbm,
               pidx_v, soff_v, wb, cb, sem):
        wid = lax.axis_index("s") * NUM_CORES + lax.axis_index("c")
        pltpu.sync_copy(pidx_hbm.at[wid], pidx_v)
        pltpu.sync_copy(soff_hbm.at[wid], soff_v)
        ivec = lax.iota(jnp.int32, LANES)

        def chunk_body(c, carry):
            pltpu.async_copy(
                tpair_hbm.at[pidx_v.at[pl.ds(c * CHUNK, CHUNK)]], wb,
                sem).wait()

            def word_body(w, carry2):
                svec = plsc.load_gather(
                    soff_v, [jnp.full((LANES,), c * CHUNK + w, jnp.int32)])
                wvec = jnp.full((LANES,), w, jnp.int32)
                cvec0 = svec + ivec
                for k in range(19):
                    off = 16 * k if k < 18 else EMBED_DIM - LANES
                    vals = plsc.load_gather(wb, [wvec, cvec0 + off])
                    plsc.store_scatter(
                        cb, [w * EMBED_DIM + off + ivec], vals)
                return carry2

            lax.fori_loop(0, CHUNK, word_body, 0)
            base = (wid * b_per_w + c * CHUNK) * EMBED_DIM
            pltpu.sync_copy(cb, out_hbm.at[pl.ds(base, CHUNK * EMBED_DIM)])
            return carry

        lax.fori_loop(0, n_chunks, chunk_body, 0)

    return gather


def kernel(indices, table):
    batch, seq = indices.shape
    vocab, dim = table.shape
    total = batch * seq
    b_per_w = total // NUM_WORKERS
    n_chunks = b_per_w // CHUNK
    idx = indices.reshape(-1).astype(jnp.int32)
    pidx = (idx >> 1).reshape(NUM_WORKERS, b_per_w)
    soff = ((idx & 1) * EMBED_DIM).reshape(NUM_WORKERS, b_per_w)
    tpair = table.reshape(vocab // 2, PAIR_DIM)
    out = _make_gather(n_chunks, b_per_w, total)(pidx, soff, tpair)
    return out.reshape(batch, seq, dim)


# speedup vs baseline: 1.0793x; 1.0344x over previous
import functools

import jax
import jax.numpy as jnp
from jax import lax
from jax.experimental import pallas as pl
from jax.experimental.pallas import tpu as pltpu
from jax.experimental.pallas import tpu_sc as plsc

EMBED_DIM = 300
PAIR_DIM = 2 * EMBED_DIM
NUM_CORES = 2
NUM_SUBCORES = 16
NUM_WORKERS = NUM_CORES * NUM_SUBCORES
CHUNK = 64
NBUF = 2
LANES = 16

_mesh = plsc.VectorSubcoreMesh(core_axis_name="c", subcore_axis_name="s")


def _make_gather(n_chunks: int, b_per_w: int, total_rows: int):
    assert n_chunks % NBUF == 0 and n_chunks >= 2 * NBUF

    @functools.partial(
        pl.kernel,
        mesh=_mesh,
        out_type=jax.ShapeDtypeStruct((total_rows * EMBED_DIM,), jnp.float32),
        scratch_types=[
            pltpu.VMEM((b_per_w,), jnp.int32),
            pltpu.VMEM((b_per_w,), jnp.int32),
            pltpu.VMEM((CHUNK, PAIR_DIM), jnp.float32),
            pltpu.VMEM((CHUNK, PAIR_DIM), jnp.float32),
            pltpu.VMEM((CHUNK * EMBED_DIM,), jnp.float32),
            pltpu.VMEM((CHUNK * EMBED_DIM,), jnp.float32),
            pltpu.SemaphoreType.DMA,
            pltpu.SemaphoreType.DMA,
            pltpu.SemaphoreType.DMA,
            pltpu.SemaphoreType.DMA,
        ],
        compiler_params=pltpu.CompilerParams(
            use_tc_tiling_on_sc=False, needs_layout_passes=False),
    )
    def gather(pidx_hbm, soff_hbm, tpair_hbm, out_hbm,
               pidx_v, soff_v, wb0, wb1, cb0, cb1, gs0, gs1, ws0, ws1):
        wid = lax.axis_index("s") * NUM_CORES + lax.axis_index("c")
        pltpu.sync_copy(pidx_hbm.at[wid], pidx_v)
        pltpu.sync_copy(soff_hbm.at[wid], soff_v)
        ivec = lax.iota(jnp.int32, LANES)
        wb = (wb0, wb1)
        cb = (cb0, cb1)
        gs = (gs0, gs1)
        ws = (ws0, ws1)

        def gather_desc(c, b):
            return pltpu.make_async_copy(
                tpair_hbm.at[pidx_v.at[pl.ds(c * CHUNK, CHUNK)]], wb[b],
                gs[b])

        def write_desc(c, b):
            base = (wid * b_per_w + c * CHUNK) * EMBED_DIM
            return pltpu.make_async_copy(
                cb[b], out_hbm.at[pl.ds(base, CHUNK * EMBED_DIM)], ws[b])

        def compact(b):
            def word_body(w, carry2):
                svec = plsc.load_gather(soff_v, [jnp.full((LANES,), w, jnp.int32)])
                wvec = jnp.full((LANES,), w % CHUNK, jnp.int32)
                cvec0 = svec + ivec
                for k in range(19):
                    off = 16 * k if k < 18 else EMBED_DIM - LANES
                    vals = plsc.load_gather(wb[b], [wvec, cvec0 + off])
                    plsc.store_scatter(
                        cb[b], [(w % CHUNK) * EMBED_DIM + off + ivec], vals)
                return carry2
            return word_body

        def compact_chunk(c, b):
            lax.fori_loop(c * CHUNK, (c + 1) * CHUNK, compact(b), 0)

        for b in range(NBUF):
            gather_desc(b, b).start()

        for b in range(NBUF):
            gather_desc(b, b).wait()
            compact_chunk(b, b)
            gather_desc(b + NBUF, b).start()
            write_desc(b, b).start()

        def loop_body(i, carry):
            c2 = NBUF * i + NBUF
            for b in range(NBUF):
                c = c2 + b
                gather_desc(c, b).wait()
                write_desc(c - NBUF, b).wait()
                compact_chunk(c, b)
                gc = jnp.minimum(c + NBUF, n_chunks - 1)
                gather_desc(gc, b).start()
                write_desc(c, b).start()
            return carry

        lax.fori_loop(0, (n_chunks - NBUF) // NBUF, loop_body, 0)

        for b in range(NBUF):
            gather_desc(n_chunks - 1, b).wait()
            write_desc(n_chunks - NBUF + b, b).wait()

    return gather


def kernel(indices, table):
    batch, seq = indices.shape
    vocab, dim = table.shape
    total = batch * seq
    b_per_w = total // NUM_WORKERS
    n_chunks = b_per_w // CHUNK
    idx = indices.reshape(-1).astype(jnp.int32)
    pidx = (idx >> 1).reshape(NUM_WORKERS, b_per_w)
    soff = ((idx & 1) * EMBED_DIM).reshape(NUM_WORKERS, b_per_w)
    table2 = table.reshape(vocab // 2, PAIR_DIM)
    out = _make_gather(n_chunks, b_per_w, total)(pidx, soff, table2)
    return out.reshape(batch, seq, dim)


# speedup vs baseline: 1.0796x; 1.0003x over previous
import functools

import jax
import jax.numpy as jnp
from jax import lax
from jax.experimental import pallas as pl
from jax.experimental.pallas import tpu as pltpu
from jax.experimental.pallas import tpu_sc as plsc

EMBED_DIM = 300
PAIR_DIM = 2 * EMBED_DIM
NUM_CORES = 2
NUM_SUBCORES = 16
NUM_WORKERS = NUM_CORES * NUM_SUBCORES
CHUNK = 32
NBUF = 4
LANES = 16

_mesh = plsc.VectorSubcoreMesh(core_axis_name="c", subcore_axis_name="s")


def _make_gather(n_chunks: int, b_per_w: int, total_rows: int):
    assert n_chunks % NBUF == 0 and n_chunks >= 2 * NBUF

    @functools.partial(
        pl.kernel,
        mesh=_mesh,
        out_type=jax.ShapeDtypeStruct((total_rows * EMBED_DIM,), jnp.float32),
        scratch_types=(
            [pltpu.VMEM((b_per_w,), jnp.int32),
             pltpu.VMEM((b_per_w,), jnp.int32)]
            + [pltpu.VMEM((CHUNK, PAIR_DIM), jnp.float32)] * NBUF
            + [pltpu.VMEM((CHUNK * EMBED_DIM,), jnp.float32)] * NBUF
            + [pltpu.SemaphoreType.DMA] * (2 * NBUF)
        ),
        compiler_params=pltpu.CompilerParams(
            use_tc_tiling_on_sc=False, needs_layout_passes=False),
    )
    def gather(pidx_hbm, soff_hbm, tpair_hbm, out_hbm,
               pidx_v, soff_v, *scr):
        wid = lax.axis_index("s") * NUM_CORES + lax.axis_index("c")
        pltpu.sync_copy(pidx_hbm.at[wid], pidx_v)
        pltpu.sync_copy(soff_hbm.at[wid], soff_v)
        ivec = lax.iota(jnp.int32, LANES)
        wb = scr[0:NBUF]
        cb = scr[NBUF:2 * NBUF]
        gs = scr[2 * NBUF:3 * NBUF]
        ws = scr[3 * NBUF:4 * NBUF]

        def gather_desc(c, b):
            return pltpu.make_async_copy(
                tpair_hbm.at[pidx_v.at[pl.ds(c * CHUNK, CHUNK)]], wb[b],
                gs[b])

        def write_desc(c, b):
            base = (wid * b_per_w + c * CHUNK) * EMBED_DIM
            return pltpu.make_async_copy(
                cb[b], out_hbm.at[pl.ds(base, CHUNK * EMBED_DIM)], ws[b])

        def compact(b):
            def word_body(w, carry2):
                svec = plsc.load_gather(soff_v, [jnp.full((LANES,), w, jnp.int32)])
                wvec = jnp.full((LANES,), w % CHUNK, jnp.int32)
                cvec0 = svec + ivec
                for k in range(19):
                    off = 16 * k if k < 18 else EMBED_DIM - LANES
                    vals = plsc.load_gather(wb[b], [wvec, cvec0 + off])
                    plsc.store_scatter(
                        cb[b], [(w % CHUNK) * EMBED_DIM + off + ivec], vals)
                return carry2
            return word_body

        def compact_chunk(c, b):
            lax.fori_loop(c * CHUNK, (c + 1) * CHUNK, compact(b), 0)

        for b in range(NBUF):
            gather_desc(b, b).start()

        for b in range(NBUF):
            gather_desc(b, b).wait()
            compact_chunk(b, b)
            gather_desc(b + NBUF, b).start()
            write_desc(b, b).start()

        def loop_body(i, carry):
            c2 = NBUF * i + NBUF
            for b in range(NBUF):
                c = c2 + b
                gather_desc(c, b).wait()
                write_desc(c - NBUF, b).wait()
                compact_chunk(c, b)
                gc = jnp.minimum(c + NBUF, n_chunks - 1)
                gather_desc(gc, b).start()
                write_desc(c, b).start()
            return carry

        lax.fori_loop(0, (n_chunks - NBUF) // NBUF, loop_body, 0)

        for b in range(NBUF):
            gather_desc(n_chunks - 1, b).wait()
            write_desc(n_chunks - NBUF + b, b).wait()

    return gather


def kernel(indices, table):
    batch, seq = indices.shape
    vocab, dim = table.shape
    total = batch * seq
    b_per_w = total // NUM_WORKERS
    n_chunks = b_per_w // CHUNK
    idx = indices.reshape(-1).astype(jnp.int32)
    pidx = (idx >> 1).reshape(NUM_WORKERS, b_per_w)
    soff = ((idx & 1) * EMBED_DIM).reshape(NUM_WORKERS, b_per_w)
    table2 = table.reshape(vocab // 2, PAIR_DIM)
    out = _make_gather(n_chunks, b_per_w, total)(pidx, soff, table2)
    return out.reshape(batch, seq, dim)
